# baseline (speedup 1.0000x reference)
import functools

import jax
import jax.numpy as jnp
from jax import lax
from jax.experimental import pallas as pl
from jax.experimental.pallas import tpu as pltpu
from jax.experimental.pallas import tpu_sc as plsc

N = 10000
E = 320000
D = 128
NC = 2
NS = 16
NW = NC * NS
K = 128
EPT = E // NW
CF = EPT // K
TAIL = EPT - CF * K
NPAD = 10240
ZR = NPAD // NS

_mesh = plsc.VectorSubcoreMesh(core_axis_name="c", subcore_axis_name="s",
                               num_cores=NC, num_subcores=NS)


_HDEPTH = 4


@functools.partial(
    pl.kernel,
    out_type=jax.ShapeDtypeStruct((NC, NPAD), jnp.float32),
    mesh=_mesh,
    scratch_types=[
        pltpu.VMEM_SHARED((NPAD,), jnp.float32),
        pltpu.VMEM((_HDEPTH, K), jnp.int32),
        pltpu.VMEM((TAIL,), jnp.int32),
        pltpu.VMEM((K,), jnp.float32),
        pltpu.VMEM((ZR,), jnp.float32),
        [pltpu.SemaphoreType.DMA] * _HDEPTH,
        [pltpu.SemaphoreType.DMA] * _HDEPTH,
    ],
)
def _hist_k(dst_hbm, out_hbm, hist_sh, dst_v, dst_tail, ones_v, zer_v, isems,
            ssems):
    c = lax.axis_index("c")
    s = lax.axis_index("s")
    wid = s * NC + c
    base = wid * EPT

    def idx_load(b, i):
        pltpu.async_copy(dst_hbm.at[pl.ds(base + i * K, K)], dst_v.at[b],
                         isems[b])

    def idx_drain(b):
        pltpu.make_async_copy(dst_hbm.at[pl.ds(0, K)], dst_v.at[b],
                              isems[b]).wait()

    def scatter(b):
        pltpu.async_copy(ones_v, hist_sh.at[dst_v.at[b]], ssems[b], add=True)

    def scatter_drain(b):
        pltpu.make_async_copy(ones_v, hist_sh.at[dst_v.at[0]],
                              ssems[b]).wait()

    for b in range(_HDEPTH):
        idx_load(b, b)
    for i in range(ZR // 16):
        zer_v[pl.ds(i * 16, 16)] = jnp.zeros((16,), jnp.float32)
    for i in range(K // 16):
        ones_v[pl.ds(i * 16, 16)] = jnp.ones((16,), jnp.float32)
    pltpu.sync_copy(zer_v, hist_sh.at[pl.ds(s * ZR, ZR)])
    plsc.subcore_barrier()

    def body(k, carry):
        for b in range(_HDEPTH):
            i = k * _HDEPTH + b
            idx_drain(b)
            scatter(b)

            @pl.when(i + _HDEPTH < CF)
            def _():
                scatter_drain(b)
                idx_load(b, i + _HDEPTH)
        return carry

    lax.fori_loop(0, CF // _HDEPTH, body, 0)
    for j in range((CF // _HDEPTH) * _HDEPTH, CF):
        b = j % _HDEPTH
        idx_drain(b)
        scatter(b)
    for b in range(_HDEPTH):
        scatter_drain(b)
    pltpu.sync_copy(dst_hbm.at[pl.ds(base + CF * K, TAIL)], dst_tail)
    pltpu.sync_copy(ones_v.at[pl.ds(0, TAIL)], hist_sh.at[dst_tail], add=True)
    plsc.subcore_barrier()
    pltpu.sync_copy(hist_sh.at[pl.ds(s * ZR, ZR)], out_hbm.at[c, pl.ds(s * ZR, ZR)])


_DEPTH = 2


@functools.partial(
    pl.kernel,
    out_type=jax.ShapeDtypeStruct((NC, NPAD, D), jnp.float32),
    mesh=_mesh,
    scratch_types=[
        pltpu.VMEM_SHARED((NPAD, D), jnp.float32),
        pltpu.VMEM((_DEPTH, K), jnp.int32),
        pltpu.VMEM((_DEPTH, K), jnp.int32),
        pltpu.VMEM((_DEPTH, K, D), jnp.float32),
        pltpu.VMEM((TAIL,), jnp.int32),
        pltpu.VMEM((TAIL,), jnp.int32),
        pltpu.VMEM((TAIL, D), jnp.float32),
        pltpu.VMEM((16, D), jnp.float32),
        [pltpu.SemaphoreType.DMA] * _DEPTH,
        [pltpu.SemaphoreType.DMA] * _DEPTH,
        [pltpu.SemaphoreType.DMA] * _DEPTH,
    ],
)
def _scatter_k(g_hbm, src_hbm, dst_hbm, out_hbm, acc_sh, src_v, dst_v, rows_v,
               src_t, dst_t, rows_t, zer_v, gsems, isems, ssems):
    c = lax.axis_index("c")
    s = lax.axis_index("s")
    wid = s * NC + c
    base = wid * EPT

    def idx_load(b, i):
        pltpu.async_copy(src_hbm.at[pl.ds(base + i * K, K)], src_v.at[b],
                         isems[b])
        pltpu.async_copy(dst_hbm.at[pl.ds(base + i * K, K)], dst_v.at[b],
                         isems[b])

    def idx_drain(b):
        pltpu.make_async_copy(src_hbm.at[pl.ds(0, K)], src_v.at[b],
                              isems[b]).wait()
        pltpu.make_async_copy(dst_hbm.at[pl.ds(0, K)], dst_v.at[b],
                              isems[b]).wait()

    def gather(b):
        pltpu.async_copy(g_hbm.at[src_v.at[b]], rows_v.at[b], gsems[b])

    def gather_drain(b):
        pltpu.make_async_copy(g_hbm.at[src_v.at[0]], rows_v.at[b],
                              gsems[b]).wait()

    def scatter(b):
        pltpu.async_copy(rows_v.at[b], acc_sh.at[dst_v.at[b]], ssems[b],
                         add=True)

    def scatter_drain(b):
        pltpu.make_async_copy(rows_v.at[b], acc_sh.at[dst_v.at[0]],
                              ssems[b]).wait()

    for b in range(_DEPTH):
        idx_load(b, b)
    for r in range(16):
        for l in range(D // 16):
            zer_v[r, pl.ds(l * 16, 16)] = jnp.zeros((16,), jnp.float32)

    def zero_body(i, carry):
        pltpu.sync_copy(zer_v, acc_sh.at[pl.ds(s * ZR + i * 16, 16), :])
        return carry

    lax.fori_loop(0, ZR // 16, zero_body, 0)
    idx_drain(0)
    gather(0)
    plsc.subcore_barrier()

    def body(k, carry):
        for b in range(_DEPTH):
            i = k * _DEPTH + b
            nb = (b + 1) % _DEPTH

            @pl.when(i + 1 < CF)
            def _():
                idx_drain(nb)

                @pl.when(i >= 1)
                def _():
                    scatter_drain(nb)

                gather(nb)

            gather_drain(b)
            scatter(b)

            @pl.when(i + _DEPTH < CF)
            def _():
                idx_load(b, i + _DEPTH)
        return carry

    lax.fori_loop(0, CF // _DEPTH, body, 0)
    for b in range(_DEPTH):
        scatter_drain(b)
    pltpu.sync_copy(src_hbm.at[pl.ds(base + CF * K, TAIL)], src_t)
    pltpu.sync_copy(dst_hbm.at[pl.ds(base + CF * K, TAIL)], dst_t)
    pltpu.async_copy(g_hbm.at[src_t], rows_t, gsems[0]).wait()
    pltpu.sync_copy(rows_t, acc_sh.at[dst_t], add=True)
    plsc.subcore_barrier()
    r0 = s * ZR
    pltpu.sync_copy(acc_sh.at[pl.ds(r0, ZR), :],
                    out_hbm.at[c, pl.ds(r0, ZR), :])


_BR = 2000
_GRID = N // _BR


def _hp_specs():
    return [
        pl.BlockSpec((1, _BR, 1), lambda i: (0, i, 0)),
        pl.BlockSpec((1, _BR, 1), lambda i: (1, i, 0)),
    ]


def _dv(hpa_ref, hpb_ref):
    return lax.rsqrt(hpa_ref[0] + hpb_ref[0] + 1.0)


def _l0_body(x_ref, w_ref, hpa_ref, hpb_ref, o_ref):
    o_ref[...] = _dv(hpa_ref, hpb_ref) * jnp.dot(
        x_ref[...], w_ref[...], preferred_element_type=jnp.float32)


def _layer0(x, W0, hp):
    return pl.pallas_call(
        _l0_body,
        grid=(_GRID,),
        in_specs=[
            pl.BlockSpec((_BR, D), lambda i: (i, 0)),
            pl.BlockSpec((D, D), lambda i: (0, 0)),
        ] + _hp_specs(),
        out_specs=pl.BlockSpec((_BR, D), lambda i: (i, 0)),
        out_shape=jax.ShapeDtypeStruct((N, D), jnp.float32),
    )(x, W0, hp, hp)


def _l1_body(pa_ref, pb_ref, g_ref, hpa_ref, hpb_ref, b_ref, w_ref, o_ref):
    dv = _dv(hpa_ref, hpb_ref)
    h = dv * (pa_ref[0] + pb_ref[0] + g_ref[...]) + b_ref[...]
    h = jnp.where(h > 0, h, 0.01 * h)
    o_ref[...] = dv * jnp.dot(h, w_ref[...], preferred_element_type=jnp.float32)


def _layer1(p, g0, hp, b0r, W1):
    return pl.pallas_call(
        _l1_body,
        grid=(_GRID,),
        in_specs=[
            pl.BlockSpec((1, _BR, D), lambda i: (0, i, 0)),
            pl.BlockSpec((1, _BR, D), lambda i: (1, i, 0)),
            pl.BlockSpec((_BR, D), lambda i: (i, 0)),
        ] + _hp_specs() + [
            pl.BlockSpec((1, D), lambda i: (0, 0)),
            pl.BlockSpec((D, D), lambda i: (0, 0)),
        ],
        out_specs=pl.BlockSpec((_BR, D), lambda i: (i, 0)),
        out_shape=jax.ShapeDtypeStruct((N, D), jnp.float32),
    )(p, p, g0, hp, hp, b0r, W1)


def _fin_body(pa_ref, pb_ref, g_ref, hpa_ref, hpb_ref, b_ref, o_ref):
    o_ref[...] = _dv(hpa_ref, hpb_ref) * (
        pa_ref[0] + pb_ref[0] + g_ref[...]) + b_ref[...]


def _final(p, g1, hp, b1r):
    return pl.pallas_call(
        _fin_body,
        grid=(_GRID,),
        in_specs=[
            pl.BlockSpec((1, _BR, D), lambda i: (0, i, 0)),
            pl.BlockSpec((1, _BR, D), lambda i: (1, i, 0)),
            pl.BlockSpec((_BR, D), lambda i: (i, 0)),
        ] + _hp_specs() + [
            pl.BlockSpec((1, D), lambda i: (0, 0)),
        ],
        out_specs=pl.BlockSpec((_BR, D), lambda i: (i, 0)),
        out_shape=jax.ShapeDtypeStruct((N, D), jnp.float32),
    )(p, p, g1, hp, hp, b1r)


def kernel(x, edge_index, W0, b0, W1, b1):
    ei = edge_index.astype(jnp.int32)
    srcv, dstv = ei[0], ei[1]

    hist_p = _hist_k(dstv)
    hp = hist_p.reshape(2, NPAD, 1)

    g0 = _layer0(x, W0, hp)
    p0 = _scatter_k(g0, srcv, dstv)
    g1 = _layer1(p0, g0, hp, b0.reshape(1, D), W1)
    p1 = _scatter_k(g1, srcv, dstv)
    return _final(p1, g1, hp, b1.reshape(1, D))


# speedup vs baseline: 1.0459x; 1.0459x over previous
import functools

import jax
import jax.numpy as jnp
from jax import lax
from jax.experimental import pallas as pl
from jax.experimental.pallas import tpu as pltpu
from jax.experimental.pallas import tpu_sc as plsc

N = 10000
E = 320000
D = 128
NC = 2
NS = 16
NW = NC * NS
K = 128
NCHUNK = E // K
CF = NCHUNK // NW
XTRA = NCHUNK - CF * NW
NPAD = 10240
ZR = NPAD // NS


def _chunk_base(wid):
    return CF * wid + jnp.minimum(wid, XTRA)


def _n_chunks(wid):
    return CF + (wid < XTRA).astype(jnp.int32)

_mesh = plsc.VectorSubcoreMesh(core_axis_name="c", subcore_axis_name="s",
                               num_cores=NC, num_subcores=NS)


_HDEPTH = 4


@functools.partial(
    pl.kernel,
    out_type=jax.ShapeDtypeStruct((NC, NPAD), jnp.float32),
    mesh=_mesh,
    scratch_types=[
        pltpu.VMEM_SHARED((NPAD,), jnp.float32),
        pltpu.VMEM((_HDEPTH, 2, K), jnp.int32),
        pltpu.VMEM((_HDEPTH, K), jnp.int32),
        pltpu.VMEM((K,), jnp.float32),
        pltpu.VMEM((ZR,), jnp.float32),
        [pltpu.SemaphoreType.DMA] * _HDEPTH,
        [pltpu.SemaphoreType.DMA] * _HDEPTH,
    ],
)
def _hist_k(ei_hbm, out_hbm, hist_sh, idx_v, dst_v, ones_v, zer_v, isems,
            ssems):
    c = lax.axis_index("c")
    s = lax.axis_index("s")
    wid = s * NC + c
    cb = _chunk_base(wid)
    ncf = _n_chunks(wid)

    def idx_load(b, i):
        pltpu.async_copy(ei_hbm.at[:, pl.ds((cb + i) * K, K)], idx_v.at[b],
                         isems[b])

    def idx_drain(b):
        pltpu.make_async_copy(ei_hbm.at[:, pl.ds(0, K)], idx_v.at[b],
                              isems[b]).wait()

    def copy_dst(b):
        for l in range(K // 16):
            dst_v[b, pl.ds(l * 16, 16)] = idx_v[b, 1, pl.ds(l * 16, 16)]

    def scatter(b):
        pltpu.async_copy(ones_v, hist_sh.at[dst_v.at[b]], ssems[b], add=True)

    def scatter_drain(b):
        pltpu.make_async_copy(ones_v, hist_sh.at[dst_v.at[0]],
                              ssems[b]).wait()

    for b in range(_HDEPTH):
        idx_load(b, b)
    for i in range(ZR // 16):
        zer_v[pl.ds(i * 16, 16)] = jnp.zeros((16,), jnp.float32)
    for i in range(K // 16):
        ones_v[pl.ds(i * 16, 16)] = jnp.ones((16,), jnp.float32)
    pltpu.sync_copy(zer_v, hist_sh.at[pl.ds(s * ZR, ZR)])
    plsc.subcore_barrier()

    def body(k, carry):
        for b in range(_HDEPTH):
            i = k * _HDEPTH + b
            idx_drain(b)
            copy_dst(b)
            scatter(b)

            @pl.when(i + _HDEPTH < ncf)
            def _():
                scatter_drain(b)
                idx_load(b, i + _HDEPTH)
        return carry

    lax.fori_loop(0, CF // _HDEPTH, body, 0)
    for j in range((CF // _HDEPTH) * _HDEPTH, CF):
        b = j % _HDEPTH
        idx_drain(b)
        copy_dst(b)
        scatter(b)

    @pl.when(ncf > CF)
    def _():
        b = CF % _HDEPTH
        idx_drain(b)
        copy_dst(b)
        scatter(b)

    for b in range(_HDEPTH):
        scatter_drain(b)
    plsc.subcore_barrier()
    pltpu.sync_copy(hist_sh.at[pl.ds(s * ZR, ZR)], out_hbm.at[c, pl.ds(s * ZR, ZR)])


_DEPTH = 2


@functools.partial(
    pl.kernel,
    out_type=jax.ShapeDtypeStruct((NC, NPAD, D), jnp.float32),
    mesh=_mesh,
    scratch_types=[
        pltpu.VMEM_SHARED((NPAD, D), jnp.float32),
        pltpu.VMEM((_DEPTH, 2, K), jnp.int32),
        pltpu.VMEM((_DEPTH, K), jnp.int32),
        pltpu.VMEM((_DEPTH, K, D), jnp.float32),
        pltpu.VMEM((16, D), jnp.float32),
        [pltpu.SemaphoreType.DMA] * _DEPTH,
        [pltpu.SemaphoreType.DMA] * _DEPTH,
        [pltpu.SemaphoreType.DMA] * _DEPTH,
    ],
)
def _scatter_k(g_hbm, ei_hbm, out_hbm, acc_sh, idx_v, dst_v, rows_v,
               zer_v, gsems, isems, ssems):
    c = lax.axis_index("c")
    s = lax.axis_index("s")
    wid = s * NC + c
    cb = _chunk_base(wid)
    ncf = _n_chunks(wid)

    def idx_load(b, i):
        pltpu.async_copy(ei_hbm.at[:, pl.ds((cb + i) * K, K)], idx_v.at[b],
                         isems[b])

    def idx_drain(b):
        pltpu.make_async_copy(ei_hbm.at[:, pl.ds(0, K)], idx_v.at[b],
                              isems[b]).wait()

    def copy_dst(b):
        for l in range(K // 16):
            dst_v[b, pl.ds(l * 16, 16)] = idx_v[b, 1, pl.ds(l * 16, 16)]

    def gather(b):
        pltpu.async_copy(g_hbm.at[idx_v.at[b, 0]], rows_v.at[b], gsems[b])

    def gather_drain(b):
        pltpu.make_async_copy(g_hbm.at[idx_v.at[0, 0]], rows_v.at[b],
                              gsems[b]).wait()

    def scatter(b):
        pltpu.async_copy(rows_v.at[b], acc_sh.at[dst_v.at[b]], ssems[b],
                         add=True)

    def scatter_drain(b):
        pltpu.make_async_copy(rows_v.at[b], acc_sh.at[dst_v.at[0]],
                              ssems[b]).wait()

    for b in range(_DEPTH):
        idx_load(b, b)
    for r in range(16):
        for l in range(D // 16):
            zer_v[r, pl.ds(l * 16, 16)] = jnp.zeros((16,), jnp.float32)

    def zero_body(i, carry):
        pltpu.sync_copy(zer_v, acc_sh.at[pl.ds(s * ZR + i * 16, 16), :])
        return carry

    lax.fori_loop(0, ZR // 16, zero_body, 0)
    idx_drain(0)
    copy_dst(0)
    gather(0)
    plsc.subcore_barrier()

    def body(k, carry):
        for b in range(_DEPTH):
            i = k * _DEPTH + b
            nb = (b + 1) % _DEPTH

            @pl.when(i + 1 < ncf)
            def _():
                idx_drain(nb)

                @pl.when(i >= 1)
                def _():
                    scatter_drain(nb)

                copy_dst(nb)
                gather(nb)

            gather_drain(b)
            scatter(b)

            @pl.when(i + _DEPTH < ncf)
            def _():
                idx_load(b, i + _DEPTH)
        return carry

    lax.fori_loop(0, CF // _DEPTH, body, 0)

    @pl.when(ncf > CF)
    def _():
        gather_drain(CF % _DEPTH)
        scatter(CF % _DEPTH)

    for b in range(_DEPTH):
        scatter_drain(b)
    plsc.subcore_barrier()
    r0 = s * ZR
    pltpu.sync_copy(acc_sh.at[pl.ds(r0, ZR), :],
                    out_hbm.at[c, pl.ds(r0, ZR), :])


_BR = 2000
_GRID = N // _BR


def _hp_specs():
    return [
        pl.BlockSpec((1, _BR, 1), lambda i: (0, i, 0)),
        pl.BlockSpec((1, _BR, 1), lambda i: (1, i, 0)),
    ]


def _dv(hpa_ref, hpb_ref):
    return lax.rsqrt(hpa_ref[0] + hpb_ref[0] + 1.0)


def _l0_body(x_ref, w_ref, hpa_ref, hpb_ref, o_ref):
    o_ref[...] = _dv(hpa_ref, hpb_ref) * jnp.dot(
        x_ref[...], w_ref[...], preferred_element_type=jnp.float32)


def _layer0(x, W0, hp):
    return pl.pallas_call(
        _l0_body,
        grid=(_GRID,),
        in_specs=[
            pl.BlockSpec((_BR, D), lambda i: (i, 0)),
            pl.BlockSpec((D, D), lambda i: (0, 0)),
        ] + _hp_specs(),
        out_specs=pl.BlockSpec((_BR, D), lambda i: (i, 0)),
        out_shape=jax.ShapeDtypeStruct((N, D), jnp.float32),
    )(x, W0, hp, hp)


def _l1_body(pa_ref, pb_ref, g_ref, hpa_ref, hpb_ref, b_ref, w_ref, o_ref):
    dv = _dv(hpa_ref, hpb_ref)
    h = dv * (pa_ref[0] + pb_ref[0] + g_ref[...]) + b_ref[...]
    h = jnp.where(h > 0, h, 0.01 * h)
    o_ref[...] = dv * jnp.dot(h, w_ref[...], preferred_element_type=jnp.float32)


def _layer1(p, g0, hp, b0r, W1):
    return pl.pallas_call(
        _l1_body,
        grid=(_GRID,),
        in_specs=[
            pl.BlockSpec((1, _BR, D), lambda i: (0, i, 0)),
            pl.BlockSpec((1, _BR, D), lambda i: (1, i, 0)),
            pl.BlockSpec((_BR, D), lambda i: (i, 0)),
        ] + _hp_specs() + [
            pl.BlockSpec((1, D), lambda i: (0, 0)),
            pl.BlockSpec((D, D), lambda i: (0, 0)),
        ],
        out_specs=pl.BlockSpec((_BR, D), lambda i: (i, 0)),
        out_shape=jax.ShapeDtypeStruct((N, D), jnp.float32),
    )(p, p, g0, hp, hp, b0r, W1)


def _fin_body(pa_ref, pb_ref, g_ref, hpa_ref, hpb_ref, b_ref, o_ref):
    o_ref[...] = _dv(hpa_ref, hpb_ref) * (
        pa_ref[0] + pb_ref[0] + g_ref[...]) + b_ref[...]


def _final(p, g1, hp, b1r):
    return pl.pallas_call(
        _fin_body,
        grid=(_GRID,),
        in_specs=[
            pl.BlockSpec((1, _BR, D), lambda i: (0, i, 0)),
            pl.BlockSpec((1, _BR, D), lambda i: (1, i, 0)),
            pl.BlockSpec((_BR, D), lambda i: (i, 0)),
        ] + _hp_specs() + [
            pl.BlockSpec((1, D), lambda i: (0, 0)),
        ],
        out_specs=pl.BlockSpec((_BR, D), lambda i: (i, 0)),
        out_shape=jax.ShapeDtypeStruct((N, D), jnp.float32),
    )(p, p, g1, hp, hp, b1r)


def kernel(x, edge_index, W0, b0, W1, b1):
    ei = edge_index.astype(jnp.int32)

    hist_p = _hist_k(ei)
    hp = hist_p.reshape(2, NPAD, 1)

    g0 = _layer0(x, W0, hp)
    p0 = _scatter_k(g0, ei)
    g1 = _layer1(p0, g0, hp, b0.reshape(1, D), W1)
    p1 = _scatter_k(g1, ei)
    return _final(p1, g1, hp, b1.reshape(1, D))


# speedup vs baseline: 1.0895x; 1.0417x over previous
import functools

import jax
import jax.numpy as jnp
from jax import lax
from jax.experimental import pallas as pl
from jax.experimental.pallas import tpu as pltpu
from jax.experimental.pallas import tpu_sc as plsc

N = 10000
E = 320000
D = 128
NC = 2
NS = 16
NW = NC * NS
K = 128
NCHUNK = E // K
CF = NCHUNK // NW
XTRA = NCHUNK - CF * NW
NPAD = 10240
ZR = NPAD // NS


def _chunk_base(wid):
    return CF * wid + jnp.minimum(wid, XTRA)


def _n_chunks(wid):
    return CF + (wid < XTRA).astype(jnp.int32)

_mesh = plsc.VectorSubcoreMesh(core_axis_name="c", subcore_axis_name="s",
                               num_cores=NC, num_subcores=NS)


_HDEPTH = 4


@functools.partial(
    pl.kernel,
    out_type=jax.ShapeDtypeStruct((NC, NPAD), jnp.float32),
    mesh=_mesh,
    scratch_types=[
        pltpu.VMEM_SHARED((NPAD,), jnp.float32),
        pltpu.VMEM((2, 2, _HDEPTH * K), jnp.int32),
        pltpu.VMEM((_HDEPTH, K), jnp.int32),
        pltpu.VMEM((K,), jnp.float32),
        pltpu.VMEM((ZR,), jnp.float32),
        [pltpu.SemaphoreType.DMA] * 2,
        [pltpu.SemaphoreType.DMA] * _HDEPTH,
    ],
)
def _hist_k(ei_hbm, out_hbm, hist_sh, idx_v, dst_v, ones_v, zer_v, isems,
            ssems):
    c = lax.axis_index("c")
    s = lax.axis_index("s")
    wid = s * NC + c
    cb = _chunk_base(wid)
    ncf = _n_chunks(wid)
    SG = _HDEPTH * K

    def idx_load(u, i):
        pltpu.async_copy(ei_hbm.at[:, pl.ds((cb + i * _HDEPTH) * K, SG)],
                         idx_v.at[u], isems[u])

    def idx_drain(u):
        pltpu.make_async_copy(ei_hbm.at[:, pl.ds(0, SG)], idx_v.at[u],
                              isems[u]).wait()

    def copy_dst(u, b):
        for l in range(K // 16):
            dst_v[b, pl.ds(l * 16, 16)] = idx_v[u, 1,
                                                pl.ds(b * K + l * 16, 16)]

    def scatter(b):
        pltpu.async_copy(ones_v, hist_sh.at[dst_v.at[b]], ssems[b], add=True)

    def scatter_drain(b):
        pltpu.make_async_copy(ones_v, hist_sh.at[dst_v.at[0]],
                              ssems[b]).wait()

    NSG = CF // _HDEPTH
    idx_load(0, 0)
    idx_load(1, 1)
    for i in range(ZR // 16):
        zer_v[pl.ds(i * 16, 16)] = jnp.zeros((16,), jnp.float32)
    for i in range(K // 16):
        ones_v[pl.ds(i * 16, 16)] = jnp.ones((16,), jnp.float32)
    pltpu.sync_copy(zer_v, hist_sh.at[pl.ds(s * ZR, ZR)])
    plsc.subcore_barrier()

    def body(kk, carry):
        for u in range(2):
            k2 = kk * 2 + u
            idx_drain(u)
            for b in range(_HDEPTH):
                if u == 0:
                    @pl.when(k2 > 0)
                    def _():
                        scatter_drain(b)
                else:
                    scatter_drain(b)
                copy_dst(u, b)
                scatter(b)

            @pl.when(k2 + 2 < NSG)
            def _():
                idx_load(u, k2 + 2)
        return carry

    lax.fori_loop(0, NSG // 2, body, 0)
    idx_drain(0)
    for b in range(_HDEPTH):
        scatter_drain(b)
        copy_dst(0, b)
        scatter(b)
    last = jnp.minimum(cb + NSG * _HDEPTH, NCHUNK - _HDEPTH)
    pltpu.async_copy(ei_hbm.at[:, pl.ds(last * K, SG)], idx_v.at[1], isems[1])
    idx_drain(1)
    off0 = (cb + NSG * _HDEPTH) - last
    ntail = ncf - NSG * _HDEPTH
    for t in range(2):
        scatter_drain(t)
        for l in range(K // 16):
            dst_v[t, pl.ds(l * 16, 16)] = idx_v[1, 1,
                                                pl.ds((off0 + t) * K + l * 16,
                                                      16)]
        scatter(t)

    @pl.when(ntail > 2)
    def _():
        scatter_drain(2)
        for l in range(K // 16):
            dst_v[2, pl.ds(l * 16, 16)] = idx_v[1, 1,
                                                pl.ds((off0 + 2) * K + l * 16,
                                                      16)]
        scatter(2)

    for b in range(_HDEPTH):
        scatter_drain(b)
    plsc.subcore_barrier()
    pltpu.sync_copy(hist_sh.at[pl.ds(s * ZR, ZR)], out_hbm.at[c, pl.ds(s * ZR, ZR)])


_DEPTH = 2


@functools.partial(
    pl.kernel,
    out_type=jax.ShapeDtypeStruct((NC, NPAD, D), jnp.float32),
    mesh=_mesh,
    scratch_types=[
        pltpu.VMEM_SHARED((NPAD, D), jnp.float32),
        pltpu.VMEM((_DEPTH, 2, K), jnp.int32),
        pltpu.VMEM((_DEPTH, K), jnp.int32),
        pltpu.VMEM((_DEPTH, K, D), jnp.float32),
        pltpu.VMEM((16, D), jnp.float32),
        [pltpu.SemaphoreType.DMA] * _DEPTH,
        [pltpu.SemaphoreType.DMA] * _DEPTH,
        [pltpu.SemaphoreType.DMA] * _DEPTH,
    ],
)
def _scatter_k(g_hbm, ei_hbm, out_hbm, acc_sh, idx_v, dst_v, rows_v,
               zer_v, gsems, isems, ssems):
    c = lax.axis_index("c")
    s = lax.axis_index("s")
    wid = s * NC + c
    cb = _chunk_base(wid)
    ncf = _n_chunks(wid)

    def idx_load(b, i):
        pltpu.async_copy(ei_hbm.at[:, pl.ds((cb + i) * K, K)], idx_v.at[b],
                         isems[b])

    def idx_drain(b):
        pltpu.make_async_copy(ei_hbm.at[:, pl.ds(0, K)], idx_v.at[b],
                              isems[b]).wait()

    def copy_dst(b):
        for l in range(K // 16):
            dst_v[b, pl.ds(l * 16, 16)] = idx_v[b, 1, pl.ds(l * 16, 16)]

    def gather(b):
        pltpu.async_copy(g_hbm.at[idx_v.at[b, 0]], rows_v.at[b], gsems[b])

    def gather_drain(b):
        pltpu.make_async_copy(g_hbm.at[idx_v.at[0, 0]], rows_v.at[b],
                              gsems[b]).wait()

    def scatter(b):
        pltpu.async_copy(rows_v.at[b], acc_sh.at[dst_v.at[b]], ssems[b],
                         add=True)

    def scatter_drain(b):
        pltpu.make_async_copy(rows_v.at[b], acc_sh.at[dst_v.at[0]],
                              ssems[b]).wait()

    for b in range(_DEPTH):
        idx_load(b, b)
    for r in range(16):
        for l in range(D // 16):
            zer_v[r, pl.ds(l * 16, 16)] = jnp.zeros((16,), jnp.float32)

    def zero_body(i, carry):
        pltpu.sync_copy(zer_v, acc_sh.at[pl.ds(s * ZR + i * 16, 16), :])
        return carry

    lax.fori_loop(0, ZR // 16, zero_body, 0)
    idx_drain(0)
    copy_dst(0)
    gather(0)
    plsc.subcore_barrier()

    def body(k, carry):
        for b in range(_DEPTH):
            i = k * _DEPTH + b
            nb = (b + 1) % _DEPTH

            @pl.when(i + 1 < ncf)
            def _():
                idx_drain(nb)

                @pl.when(i >= 1)
                def _():
                    scatter_drain(nb)

                copy_dst(nb)
                gather(nb)

            gather_drain(b)
            scatter(b)

            @pl.when(i + _DEPTH < ncf)
            def _():
                idx_load(b, i + _DEPTH)
        return carry

    lax.fori_loop(0, CF // _DEPTH, body, 0)

    @pl.when(ncf > CF)
    def _():
        gather_drain(CF % _DEPTH)
        scatter(CF % _DEPTH)

    for b in range(_DEPTH):
        scatter_drain(b)
    plsc.subcore_barrier()
    r0 = s * ZR
    pltpu.sync_copy(acc_sh.at[pl.ds(r0, ZR), :],
                    out_hbm.at[c, pl.ds(r0, ZR), :])


_BR = 2000
_GRID = N // _BR


def _hp_spec():
    return [pl.BlockSpec((_BR, 1), lambda i: (i, 0))]


def _dv(hp_ref):
    return lax.rsqrt(hp_ref[...] + 1.0)


def _l0_body(x_ref, w_ref, hp_ref, o_ref):
    o_ref[...] = _dv(hp_ref) * jnp.dot(
        x_ref[...], w_ref[...], preferred_element_type=jnp.float32)


def _layer0(x, W0, hp):
    return pl.pallas_call(
        _l0_body,
        grid=(_GRID,),
        in_specs=[
            pl.BlockSpec((_BR, D), lambda i: (i, 0)),
            pl.BlockSpec((D, D), lambda i: (0, 0)),
        ] + _hp_spec(),
        out_specs=pl.BlockSpec((_BR, D), lambda i: (i, 0)),
        out_shape=jax.ShapeDtypeStruct((N, D), jnp.float32),
    )(x, W0, hp)


def _l1_body(pa_ref, pb_ref, g_ref, hp_ref, b_ref, w_ref, o_ref):
    dv = _dv(hp_ref)
    h = dv * (pa_ref[0] + pb_ref[0] + g_ref[...]) + b_ref[...]
    h = jnp.where(h > 0, h, 0.01 * h)
    o_ref[...] = dv * jnp.dot(h, w_ref[...], preferred_element_type=jnp.float32)


def _layer1(p, g0, hp, b0r, W1):
    return pl.pallas_call(
        _l1_body,
        grid=(_GRID,),
        in_specs=[
            pl.BlockSpec((1, _BR, D), lambda i: (0, i, 0)),
            pl.BlockSpec((1, _BR, D), lambda i: (1, i, 0)),
            pl.BlockSpec((_BR, D), lambda i: (i, 0)),
        ] + _hp_spec() + [
            pl.BlockSpec((1, D), lambda i: (0, 0)),
            pl.BlockSpec((D, D), lambda i: (0, 0)),
        ],
        out_specs=pl.BlockSpec((_BR, D), lambda i: (i, 0)),
        out_shape=jax.ShapeDtypeStruct((N, D), jnp.float32),
    )(p, p, g0, hp, b0r, W1)


def _fin_body(pa_ref, pb_ref, g_ref, hp_ref, b_ref, o_ref):
    o_ref[...] = _dv(hp_ref) * (
        pa_ref[0] + pb_ref[0] + g_ref[...]) + b_ref[...]


def _final(p, g1, hp, b1r):
    return pl.pallas_call(
        _fin_body,
        grid=(_GRID,),
        in_specs=[
            pl.BlockSpec((1, _BR, D), lambda i: (0, i, 0)),
            pl.BlockSpec((1, _BR, D), lambda i: (1, i, 0)),
            pl.BlockSpec((_BR, D), lambda i: (i, 0)),
        ] + _hp_spec() + [
            pl.BlockSpec((1, D), lambda i: (0, 0)),
        ],
        out_specs=pl.BlockSpec((_BR, D), lambda i: (i, 0)),
        out_shape=jax.ShapeDtypeStruct((N, D), jnp.float32),
    )(p, p, g1, hp, b1r)


def kernel(x, edge_index, W0, b0, W1, b1):
    ei = edge_index.astype(jnp.int32)

    hist_p = _hist_k(ei)
    hp = (hist_p[0] + hist_p[1]).reshape(NPAD, 1)

    g0 = _layer0(x, W0, hp)
    p0 = _scatter_k(g0, ei)
    g1 = _layer1(p0, g0, hp, b0.reshape(1, D), W1)
    p1 = _scatter_k(g1, ei)
    return _final(p1, g1, hp, b1.reshape(1, D))


# speedup vs baseline: 1.1055x; 1.0147x over previous
import functools

import jax
import jax.numpy as jnp
from jax import lax
from jax.experimental import pallas as pl
from jax.experimental.pallas import tpu as pltpu
from jax.experimental.pallas import tpu_sc as plsc

N = 10000
E = 320000
D = 128
NC = 2
NS = 16
NW = NC * NS
K = 128
NCHUNK = E // K
CF = NCHUNK // NW
XTRA = NCHUNK - CF * NW
NPAD = 10240
ZR = NPAD // NS


def _chunk_base(wid):
    return CF * wid + jnp.minimum(wid, XTRA)


def _n_chunks(wid):
    return CF + (wid < XTRA).astype(jnp.int32)

_mesh = plsc.VectorSubcoreMesh(core_axis_name="c", subcore_axis_name="s",
                               num_cores=NC, num_subcores=NS)


_HDEPTH = 4


@functools.partial(
    pl.kernel,
    out_type=jax.ShapeDtypeStruct((NC, NPAD), jnp.float32),
    mesh=_mesh,
    scratch_types=[
        pltpu.VMEM_SHARED((NPAD,), jnp.float32),
        pltpu.VMEM((2, 2, _HDEPTH * K), jnp.int32),
        pltpu.VMEM((_HDEPTH, K), jnp.int32),
        pltpu.VMEM((K,), jnp.float32),
        pltpu.VMEM((ZR,), jnp.float32),
        [pltpu.SemaphoreType.DMA] * 2,
        [pltpu.SemaphoreType.DMA] * _HDEPTH,
    ],
)
def _hist_k(ei_hbm, out_hbm, hist_sh, idx_v, dst_v, ones_v, zer_v, isems,
            ssems):
    c = lax.axis_index("c")
    s = lax.axis_index("s")
    wid = s * NC + c
    cb = _chunk_base(wid)
    ncf = _n_chunks(wid)
    SG = _HDEPTH * K

    def idx_load(u, i):
        pltpu.async_copy(ei_hbm.at[:, pl.ds((cb + i * _HDEPTH) * K, SG)],
                         idx_v.at[u], isems[u])

    def idx_drain(u):
        pltpu.make_async_copy(ei_hbm.at[:, pl.ds(0, SG)], idx_v.at[u],
                              isems[u]).wait()

    def copy_dst(u, b):
        for l in range(K // 16):
            dst_v[b, pl.ds(l * 16, 16)] = idx_v[u, 1,
                                                pl.ds(b * K + l * 16, 16)]

    def scatter(b):
        pltpu.async_copy(ones_v, hist_sh.at[dst_v.at[b]], ssems[b], add=True)

    def scatter_drain(b):
        pltpu.make_async_copy(ones_v, hist_sh.at[dst_v.at[0]],
                              ssems[b]).wait()

    NSG = CF // _HDEPTH
    idx_load(0, 0)
    idx_load(1, 1)
    for i in range(ZR // 16):
        zer_v[pl.ds(i * 16, 16)] = jnp.zeros((16,), jnp.float32)
    for i in range(K // 16):
        ones_v[pl.ds(i * 16, 16)] = jnp.ones((16,), jnp.float32)
    pltpu.sync_copy(zer_v, hist_sh.at[pl.ds(s * ZR, ZR)])
    plsc.subcore_barrier()

    def body(kk, carry):
        for u in range(2):
            k2 = kk * 2 + u
            idx_drain(u)
            for b in range(_HDEPTH):
                if u == 0:
                    @pl.when(k2 > 0)
                    def _():
                        scatter_drain(b)
                else:
                    scatter_drain(b)
                copy_dst(u, b)
                scatter(b)

            @pl.when(k2 + 2 < NSG)
            def _():
                idx_load(u, k2 + 2)
        return carry

    lax.fori_loop(0, NSG // 2, body, 0)
    idx_drain(0)
    for b in range(_HDEPTH):
        scatter_drain(b)
        copy_dst(0, b)
        scatter(b)
    last = jnp.minimum(cb + NSG * _HDEPTH, NCHUNK - _HDEPTH)
    pltpu.async_copy(ei_hbm.at[:, pl.ds(last * K, SG)], idx_v.at[1], isems[1])
    idx_drain(1)
    off0 = (cb + NSG * _HDEPTH) - last
    ntail = ncf - NSG * _HDEPTH
    for t in range(2):
        scatter_drain(t)
        for l in range(K // 16):
            dst_v[t, pl.ds(l * 16, 16)] = idx_v[1, 1,
                                                pl.ds((off0 + t) * K + l * 16,
                                                      16)]
        scatter(t)

    @pl.when(ntail > 2)
    def _():
        scatter_drain(2)
        for l in range(K // 16):
            dst_v[2, pl.ds(l * 16, 16)] = idx_v[1, 1,
                                                pl.ds((off0 + 2) * K + l * 16,
                                                      16)]
        scatter(2)

    for b in range(_HDEPTH):
        scatter_drain(b)
    plsc.subcore_barrier()
    pltpu.sync_copy(hist_sh.at[pl.ds(s * ZR, ZR)], out_hbm.at[c, pl.ds(s * ZR, ZR)])


_DEPTH = 2


@functools.partial(
    pl.kernel,
    out_type=jax.ShapeDtypeStruct((NC, NPAD, D), jnp.float32),
    mesh=_mesh,
    scratch_types=[
        pltpu.VMEM_SHARED((NPAD, D), jnp.float32),
        pltpu.VMEM((_DEPTH, 2, K), jnp.int32),
        pltpu.VMEM((_DEPTH, K), jnp.int32),
        pltpu.VMEM((_DEPTH, K, D), jnp.float32),
        pltpu.VMEM((16, D), jnp.float32),
        [pltpu.SemaphoreType.DMA] * _DEPTH,
        [pltpu.SemaphoreType.DMA] * _DEPTH,
        [pltpu.SemaphoreType.DMA] * _DEPTH,
    ],
)
def _scatter_k(g_hbm, ei_hbm, out_hbm, acc_sh, idx_v, dst_v, rows_v,
               zer_v, gsems, isems, ssems):
    c = lax.axis_index("c")
    s = lax.axis_index("s")
    wid = s * NC + c
    cb = _chunk_base(wid)
    ncf = _n_chunks(wid)

    def idx_load(b, i):
        pltpu.async_copy(ei_hbm.at[:, pl.ds((cb + i) * K, K)], idx_v.at[b],
                         isems[b])

    def idx_drain(b):
        pltpu.make_async_copy(ei_hbm.at[:, pl.ds(0, K)], idx_v.at[b],
                              isems[b]).wait()

    def copy_dst(b):
        for l in range(K // 16):
            dst_v[b, pl.ds(l * 16, 16)] = idx_v[b, 1, pl.ds(l * 16, 16)]

    def gather(b):
        pltpu.async_copy(g_hbm.at[idx_v.at[b, 0]], rows_v.at[b], gsems[b])

    def gather_drain(b):
        pltpu.make_async_copy(g_hbm.at[idx_v.at[0, 0]], rows_v.at[b],
                              gsems[b]).wait()

    def scatter(b):
        pltpu.async_copy(rows_v.at[b], acc_sh.at[dst_v.at[b]], ssems[b],
                         add=True)

    def scatter_drain(b):
        pltpu.make_async_copy(rows_v.at[b], acc_sh.at[dst_v.at[0]],
                              ssems[b]).wait()

    for b in range(_DEPTH):
        idx_load(b, b)
    for r in range(16):
        for l in range(D // 16):
            zer_v[r, pl.ds(l * 16, 16)] = jnp.zeros((16,), jnp.float32)

    def zero_body(i, carry):
        pltpu.sync_copy(zer_v, acc_sh.at[pl.ds(s * ZR + i * 16, 16), :])
        return carry

    lax.fori_loop(0, ZR // 16, zero_body, 0)
    idx_drain(0)
    copy_dst(0)
    gather(0)
    plsc.subcore_barrier()

    def body(k, carry):
        for b in range(_DEPTH):
            i = k * _DEPTH + b
            nb = (b + 1) % _DEPTH

            @pl.when(i + 1 < ncf)
            def _():
                idx_drain(nb)

                @pl.when(i >= 1)
                def _():
                    scatter_drain(nb)

                copy_dst(nb)
                gather(nb)

            gather_drain(b)
            scatter(b)

            @pl.when(i + _DEPTH < ncf)
            def _():
                idx_load(b, i + _DEPTH)
        return carry

    lax.fori_loop(0, CF // _DEPTH, body, 0)

    @pl.when(ncf > CF)
    def _():
        gather_drain(CF % _DEPTH)
        scatter(CF % _DEPTH)

    for b in range(_DEPTH):
        scatter_drain(b)
    plsc.subcore_barrier()
    r0 = s * ZR
    pltpu.sync_copy(acc_sh.at[pl.ds(r0, ZR), :],
                    out_hbm.at[c, pl.ds(r0, ZR), :])


_BR = 5000
_GRID = N // _BR


def _hp_spec():
    return [pl.BlockSpec((_BR, 1), lambda i: (i, 0))]


def _dv(hp_ref):
    return lax.rsqrt(hp_ref[...] + 1.0)


def _l0_body(x_ref, w_ref, hp_ref, o_ref):
    o_ref[...] = _dv(hp_ref) * jnp.dot(
        x_ref[...], w_ref[...], preferred_element_type=jnp.float32)


def _layer0(x, W0, hp):
    return pl.pallas_call(
        _l0_body,
        grid=(_GRID,),
        in_specs=[
            pl.BlockSpec((_BR, D), lambda i: (i, 0)),
            pl.BlockSpec((D, D), lambda i: (0, 0)),
        ] + _hp_spec(),
        out_specs=pl.BlockSpec((_BR, D), lambda i: (i, 0)),
        out_shape=jax.ShapeDtypeStruct((N, D), jnp.float32),
    )(x, W0, hp)


def _l1_body(pa_ref, pb_ref, g_ref, hp_ref, b_ref, w_ref, o_ref):
    dv = _dv(hp_ref)
    h = dv * (pa_ref[0] + pb_ref[0] + g_ref[...]) + b_ref[...]
    h = jnp.where(h > 0, h, 0.01 * h)
    o_ref[...] = dv * jnp.dot(h, w_ref[...], preferred_element_type=jnp.float32)


def _layer1(p, g0, hp, b0r, W1):
    return pl.pallas_call(
        _l1_body,
        grid=(_GRID,),
        in_specs=[
            pl.BlockSpec((1, _BR, D), lambda i: (0, i, 0)),
            pl.BlockSpec((1, _BR, D), lambda i: (1, i, 0)),
            pl.BlockSpec((_BR, D), lambda i: (i, 0)),
        ] + _hp_spec() + [
            pl.BlockSpec((1, D), lambda i: (0, 0)),
            pl.BlockSpec((D, D), lambda i: (0, 0)),
        ],
        out_specs=pl.BlockSpec((_BR, D), lambda i: (i, 0)),
        out_shape=jax.ShapeDtypeStruct((N, D), jnp.float32),
    )(p, p, g0, hp, b0r, W1)


def _fin_body(pa_ref, pb_ref, g_ref, hp_ref, b_ref, o_ref):
    o_ref[...] = _dv(hp_ref) * (
        pa_ref[0] + pb_ref[0] + g_ref[...]) + b_ref[...]


def _final(p, g1, hp, b1r):
    return pl.pallas_call(
        _fin_body,
        grid=(_GRID,),
        in_specs=[
            pl.BlockSpec((1, _BR, D), lambda i: (0, i, 0)),
            pl.BlockSpec((1, _BR, D), lambda i: (1, i, 0)),
            pl.BlockSpec((_BR, D), lambda i: (i, 0)),
        ] + _hp_spec() + [
            pl.BlockSpec((1, D), lambda i: (0, 0)),
        ],
        out_specs=pl.BlockSpec((_BR, D), lambda i: (i, 0)),
        out_shape=jax.ShapeDtypeStruct((N, D), jnp.float32),
    )(p, p, g1, hp, b1r)


def kernel(x, edge_index, W0, b0, W1, b1):
    ei = edge_index.astype(jnp.int32)

    hist_p = _hist_k(ei)
    hp = (hist_p[0] + hist_p[1]).reshape(NPAD, 1)

    g0 = _layer0(x, W0, hp)
    p0 = _scatter_k(g0, ei)
    g1 = _layer1(p0, g0, hp, b0.reshape(1, D), W1)
    p1 = _scatter_k(g1, ei)
    return _final(p1, g1, hp, b1.reshape(1, D))


# speedup vs baseline: 1.1188x; 1.0121x over previous
import functools

import jax
import jax.numpy as jnp
from jax import lax
from jax.experimental import pallas as pl
from jax.experimental.pallas import tpu as pltpu
from jax.experimental.pallas import tpu_sc as plsc

N = 10000
E = 320000
D = 128
NC = 2
NS = 16
NW = NC * NS
K = 128
NCHUNK = E // K
CF = NCHUNK // NW
XTRA = NCHUNK - CF * NW
NPAD = 10240
ZR = NPAD // NS


def _chunk_base(wid):
    return CF * wid + jnp.minimum(wid, XTRA)


def _n_chunks(wid):
    return CF + (wid < XTRA).astype(jnp.int32)

_mesh = plsc.VectorSubcoreMesh(core_axis_name="c", subcore_axis_name="s",
                               num_cores=NC, num_subcores=NS)


_HDEPTH = 4


@functools.partial(
    pl.kernel,
    out_type=jax.ShapeDtypeStruct((NC, NPAD), jnp.float32),
    mesh=_mesh,
    scratch_types=[
        pltpu.VMEM_SHARED((NPAD,), jnp.float32),
        pltpu.VMEM((2, 2, _HDEPTH * K), jnp.int32),
        pltpu.VMEM((_HDEPTH, K), jnp.int32),
        pltpu.VMEM((K,), jnp.float32),
        pltpu.VMEM((ZR,), jnp.float32),
        [pltpu.SemaphoreType.DMA] * 2,
        [pltpu.SemaphoreType.DMA] * _HDEPTH,
    ],
)
def _hist_k(ei_hbm, out_hbm, hist_sh, idx_v, dst_v, ones_v, zer_v, isems,
            ssems):
    c = lax.axis_index("c")
    s = lax.axis_index("s")
    wid = s * NC + c
    cb = _chunk_base(wid)
    ncf = _n_chunks(wid)
    SG = _HDEPTH * K

    def idx_load(u, i):
        pltpu.async_copy(ei_hbm.at[:, pl.ds((cb + i * _HDEPTH) * K, SG)],
                         idx_v.at[u], isems[u])

    def idx_drain(u):
        pltpu.make_async_copy(ei_hbm.at[:, pl.ds(0, SG)], idx_v.at[u],
                              isems[u]).wait()

    def copy_dst(u, b):
        for l in range(K // 16):
            dst_v[b, pl.ds(l * 16, 16)] = idx_v[u, 1,
                                                pl.ds(b * K + l * 16, 16)]

    def scatter(b):
        pltpu.async_copy(ones_v, hist_sh.at[dst_v.at[b]], ssems[b], add=True)

    def scatter_drain(b):
        pltpu.make_async_copy(ones_v, hist_sh.at[dst_v.at[0]],
                              ssems[b]).wait()

    NSG = CF // _HDEPTH
    idx_load(0, 0)
    idx_load(1, 1)
    for i in range(ZR // 16):
        zer_v[pl.ds(i * 16, 16)] = jnp.zeros((16,), jnp.float32)
    for i in range(K // 16):
        ones_v[pl.ds(i * 16, 16)] = jnp.ones((16,), jnp.float32)
    pltpu.sync_copy(zer_v, hist_sh.at[pl.ds(s * ZR, ZR)])
    plsc.subcore_barrier()

    def body(kk, carry):
        for u in range(2):
            k2 = kk * 2 + u
            idx_drain(u)
            for b in range(_HDEPTH):
                if u == 0:
                    @pl.when(k2 > 0)
                    def _():
                        scatter_drain(b)
                else:
                    scatter_drain(b)
                copy_dst(u, b)
                scatter(b)

            @pl.when(k2 + 2 < NSG)
            def _():
                idx_load(u, k2 + 2)
        return carry

    lax.fori_loop(0, NSG // 2, body, 0)
    idx_drain(0)
    for b in range(_HDEPTH):
        scatter_drain(b)
        copy_dst(0, b)
        scatter(b)
    last = jnp.minimum(cb + NSG * _HDEPTH, NCHUNK - _HDEPTH)
    pltpu.async_copy(ei_hbm.at[:, pl.ds(last * K, SG)], idx_v.at[1], isems[1])
    idx_drain(1)
    off0 = (cb + NSG * _HDEPTH) - last
    ntail = ncf - NSG * _HDEPTH
    for t in range(2):
        scatter_drain(t)
        for l in range(K // 16):
            dst_v[t, pl.ds(l * 16, 16)] = idx_v[1, 1,
                                                pl.ds((off0 + t) * K + l * 16,
                                                      16)]
        scatter(t)

    @pl.when(ntail > 2)
    def _():
        scatter_drain(2)
        for l in range(K // 16):
            dst_v[2, pl.ds(l * 16, 16)] = idx_v[1, 1,
                                                pl.ds((off0 + 2) * K + l * 16,
                                                      16)]
        scatter(2)

    for b in range(_HDEPTH):
        scatter_drain(b)
    plsc.subcore_barrier()
    pltpu.sync_copy(hist_sh.at[pl.ds(s * ZR, ZR)], out_hbm.at[c, pl.ds(s * ZR, ZR)])


_DEPTH = 2


@functools.partial(
    pl.kernel,
    out_type=jax.ShapeDtypeStruct((NC, NPAD, D), jnp.float32),
    mesh=_mesh,
    scratch_types=[
        pltpu.VMEM_SHARED((NPAD, D), jnp.float32),
        pltpu.VMEM((_DEPTH, 2, K), jnp.int32),
        pltpu.VMEM((_DEPTH, K), jnp.int32),
        pltpu.VMEM((_DEPTH, K, D), jnp.float32),
        pltpu.VMEM((32, D), jnp.float32),
        [pltpu.SemaphoreType.DMA] * _DEPTH,
        [pltpu.SemaphoreType.DMA] * _DEPTH,
        [pltpu.SemaphoreType.DMA] * _DEPTH,
    ],
)
def _scatter_k(g_hbm, ei_hbm, out_hbm, acc_sh, idx_v, dst_v, rows_v,
               zer_v, gsems, isems, ssems):
    c = lax.axis_index("c")
    s = lax.axis_index("s")
    wid = s * NC + c
    cb = _chunk_base(wid)
    ncf = _n_chunks(wid)

    def idx_load(b, i):
        pltpu.async_copy(ei_hbm.at[:, pl.ds((cb + i) * K, K)], idx_v.at[b],
                         isems[b])

    def idx_drain(b):
        pltpu.make_async_copy(ei_hbm.at[:, pl.ds(0, K)], idx_v.at[b],
                              isems[b]).wait()

    def copy_dst(b):
        for l in range(K // 16):
            dst_v[b, pl.ds(l * 16, 16)] = idx_v[b, 1, pl.ds(l * 16, 16)]

    def gather(b):
        pltpu.async_copy(g_hbm.at[idx_v.at[b, 0]], rows_v.at[b], gsems[b])

    def gather_drain(b):
        pltpu.make_async_copy(g_hbm.at[idx_v.at[0, 0]], rows_v.at[b],
                              gsems[b]).wait()

    def scatter(b):
        pltpu.async_copy(rows_v.at[b], acc_sh.at[dst_v.at[b]], ssems[b],
                         add=True)

    def scatter_drain(b):
        pltpu.make_async_copy(rows_v.at[b], acc_sh.at[dst_v.at[0]],
                              ssems[b]).wait()

    for b in range(_DEPTH):
        idx_load(b, b)
    for r in range(32):
        for l in range(D // 16):
            zer_v[r, pl.ds(l * 16, 16)] = jnp.zeros((16,), jnp.float32)

    def zero_body(i, carry):
        descs = []
        for b in range(4):
            descs.append(pltpu.async_copy(
                zer_v, acc_sh.at[pl.ds(s * ZR + (i * 4 + b) * 32, 32), :],
                ssems[b % _DEPTH]))
        for dsc in descs:
            dsc.wait()
        return carry

    lax.fori_loop(0, ZR // 128, zero_body, 0)
    idx_drain(0)
    copy_dst(0)
    gather(0)
    plsc.subcore_barrier()

    def body(k, carry):
        for b in range(_DEPTH):
            i = k * _DEPTH + b
            nb = (b + 1) % _DEPTH

            @pl.when(i + 1 < ncf)
            def _():
                idx_drain(nb)

                @pl.when(i >= 1)
                def _():
                    scatter_drain(nb)

                copy_dst(nb)
                gather(nb)

            gather_drain(b)
            scatter(b)

            @pl.when(i + _DEPTH < ncf)
            def _():
                idx_load(b, i + _DEPTH)
        return carry

    lax.fori_loop(0, CF // _DEPTH, body, 0)

    @pl.when(ncf > CF)
    def _():
        gather_drain(CF % _DEPTH)
        scatter(CF % _DEPTH)

    for b in range(_DEPTH):
        scatter_drain(b)
    plsc.subcore_barrier()
    r0 = s * ZR
    pltpu.sync_copy(acc_sh.at[pl.ds(r0, ZR), :],
                    out_hbm.at[c, pl.ds(r0, ZR), :])


_BR = 5000
_GRID = N // _BR


def _hp_spec():
    return [pl.BlockSpec((_BR, 1), lambda i: (i, 0))]


def _dv(hp_ref):
    return lax.rsqrt(hp_ref[...] + 1.0)


def _l0_body(x_ref, w_ref, hp_ref, o_ref):
    o_ref[...] = _dv(hp_ref) * jnp.dot(
        x_ref[...], w_ref[...], preferred_element_type=jnp.float32)


def _layer0(x, W0, hp):
    return pl.pallas_call(
        _l0_body,
        grid=(_GRID,),
        in_specs=[
            pl.BlockSpec((_BR, D), lambda i: (i, 0)),
            pl.BlockSpec((D, D), lambda i: (0, 0)),
        ] + _hp_spec(),
        out_specs=pl.BlockSpec((_BR, D), lambda i: (i, 0)),
        out_shape=jax.ShapeDtypeStruct((N, D), jnp.float32),
    )(x, W0, hp)


def _l1_body(pa_ref, pb_ref, g_ref, hp_ref, b_ref, w_ref, o_ref):
    dv = _dv(hp_ref)
    h = dv * (pa_ref[0] + pb_ref[0] + g_ref[...]) + b_ref[...]
    h = jnp.where(h > 0, h, 0.01 * h)
    o_ref[...] = dv * jnp.dot(h, w_ref[...], preferred_element_type=jnp.float32)


def _layer1(p, g0, hp, b0r, W1):
    return pl.pallas_call(
        _l1_body,
        grid=(_GRID,),
        in_specs=[
            pl.BlockSpec((1, _BR, D), lambda i: (0, i, 0)),
            pl.BlockSpec((1, _BR, D), lambda i: (1, i, 0)),
            pl.BlockSpec((_BR, D), lambda i: (i, 0)),
        ] + _hp_spec() + [
            pl.BlockSpec((1, D), lambda i: (0, 0)),
            pl.BlockSpec((D, D), lambda i: (0, 0)),
        ],
        out_specs=pl.BlockSpec((_BR, D), lambda i: (i, 0)),
        out_shape=jax.ShapeDtypeStruct((N, D), jnp.float32),
    )(p, p, g0, hp, b0r, W1)


def _fin_body(pa_ref, pb_ref, g_ref, hp_ref, b_ref, o_ref):
    o_ref[...] = _dv(hp_ref) * (
        pa_ref[0] + pb_ref[0] + g_ref[...]) + b_ref[...]


def _final(p, g1, hp, b1r):
    return pl.pallas_call(
        _fin_body,
        grid=(_GRID,),
        in_specs=[
            pl.BlockSpec((1, _BR, D), lambda i: (0, i, 0)),
            pl.BlockSpec((1, _BR, D), lambda i: (1, i, 0)),
            pl.BlockSpec((_BR, D), lambda i: (i, 0)),
        ] + _hp_spec() + [
            pl.BlockSpec((1, D), lambda i: (0, 0)),
        ],
        out_specs=pl.BlockSpec((_BR, D), lambda i: (i, 0)),
        out_shape=jax.ShapeDtypeStruct((N, D), jnp.float32),
    )(p, p, g1, hp, b1r)


def kernel(x, edge_index, W0, b0, W1, b1):
    ei = edge_index.astype(jnp.int32)

    hist_p = _hist_k(ei)
    hp = (hist_p[0] + hist_p[1]).reshape(NPAD, 1)

    g0 = _layer0(x, W0, hp)
    p0 = _scatter_k(g0, ei)
    g1 = _layer1(p0, g0, hp, b0.reshape(1, D), W1)
    p1 = _scatter_k(g1, ei)
    return _final(p1, g1, hp, b1.reshape(1, D))
